# Initial kernel scaffold; baseline (speedup 1.0000x reference)
import jax
import jax.numpy as jnp
from jax.experimental import pallas as pl


def kernel(x, edge_index, W_emb, b_emb, W_ih, W_hh, b_ih, b_hh, W_gcn, b_gcn, W_fc, b_fc):
    raise NotImplementedError("write your pallas kernel here")


# speedup vs baseline: 12.7209x; 12.7209x over previous
import functools

import jax
import jax.numpy as jnp
from jax import lax
from jax.experimental import pallas as pl
from jax.experimental.pallas import tpu as pltpu
from jax.experimental.pallas import tpu_sc as plsc

N = 10000
T = 8
D_IN = 128
H = 128
D_OUT = 128
E = 320000

NC = 2
NS = 16
NW = NC * NS
K = 125
EPW = E // NW
CPW = EPW // K
NP = 10240
RPT = NP // NS
BN = 1000


def _sc_mesh():
    return plsc.VectorSubcoreMesh(core_axis_name="c", subcore_axis_name="s")


def _scat_body(hws_hbm, row_hbm, col_hbm, zeros_hbm, out_hbm,
               rowv, colv, gbuf, acc, sem):
    cid = lax.axis_index("c")
    sid = lax.axis_index("s")
    wid = cid * NS + sid
    pltpu.sync_copy(row_hbm.at[wid], rowv)
    pltpu.sync_copy(col_hbm.at[wid], colv)
    pltpu.sync_copy(zeros_hbm.at[pl.ds(sid * RPT, RPT)],
                    acc.at[pl.ds(sid * RPT, RPT)])
    plsc.subcore_barrier()

    def body(j, carry):
        pltpu.async_copy(hws_hbm.at[rowv.at[j]], gbuf, sem).wait()
        pltpu.sync_copy(gbuf, acc.at[colv.at[j]], add=True)
        return carry

    lax.fori_loop(0, CPW, body, 0)
    plsc.subcore_barrier()
    pltpu.sync_copy(acc.at[pl.ds(sid * RPT, RPT)],
                    out_hbm.at[cid, pl.ds(sid * RPT, RPT)])


def _make_scat_kernel():
    return pl.kernel(
        _scat_body,
        out_type=jax.ShapeDtypeStruct((NC, NP, H), jnp.float32),
        mesh=_sc_mesh(),
        scratch_types=[
            pltpu.VMEM((CPW, K), jnp.int32),
            pltpu.VMEM((CPW, K), jnp.int32),
            pltpu.VMEM((K, H), jnp.float32),
            pltpu.VMEM_SHARED((NP, H), jnp.float32),
            pltpu.SemaphoreType.DMA,
        ],
    )


def _dinv_from_deg(deg):
    d = deg[0, :, 0:1] + deg[1, :, 0:1] + 1.0
    return lax.rsqrt(d)


def _gru_hws(x_blk, hprev, dinv, we, be, wih, bih, whh, bhh, wg):
    dn = (((1,), (1,)), ((), ()))
    inp = lax.dot_general(x_blk, we, dn, preferred_element_type=jnp.float32) + be
    gi = lax.dot_general(inp, wih, dn, preferred_element_type=jnp.float32) + bih
    gh = lax.dot_general(hprev, whh, dn, preferred_element_type=jnp.float32) + bhh
    r = jax.nn.sigmoid(gi[:, :H] + gh[:, :H])
    z = jax.nn.sigmoid(gi[:, H:2 * H] + gh[:, H:2 * H])
    cand = jnp.tanh(gi[:, 2 * H:] + r * gh[:, 2 * H:])
    h = (1.0 - z) * cand + z * hprev
    hw = lax.dot_general(h, wg, dn, preferred_element_type=jnp.float32)
    return dinv * hw


_W_SPECS = [
    pl.BlockSpec((H, D_IN), lambda i: (0, 0)),
    pl.BlockSpec((1, H), lambda i: (0, 0)),
    pl.BlockSpec((3 * H, H), lambda i: (0, 0)),
    pl.BlockSpec((1, 3 * H), lambda i: (0, 0)),
    pl.BlockSpec((3 * H, H), lambda i: (0, 0)),
    pl.BlockSpec((1, 3 * H), lambda i: (0, 0)),
    pl.BlockSpec((H, H), lambda i: (0, 0)),
    pl.BlockSpec((1, H), lambda i: (0, 0)),
]


def _make_step0_kernel():
    def body(x_ref, deg_ref, we, be, wih, bih, whh, bhh, wg, bg, out_ref):
        dinv = _dinv_from_deg(deg_ref[...])
        hprev = jnp.zeros((BN, H), jnp.float32)
        out_ref[...] = _gru_hws(x_ref[0], hprev, dinv, we[...], be[...],
                                wih[...], bih[...], whh[...], bhh[...], wg[...])

    in_specs = [
        pl.BlockSpec((1, BN, D_IN), lambda i: (0, i, 0)),
        pl.BlockSpec((NC, BN, H), lambda i: (0, i, 0)),
    ] + _W_SPECS
    return pl.pallas_call(
        body,
        grid=(N // BN,),
        in_specs=in_specs,
        out_specs=pl.BlockSpec((BN, H), lambda i: (i, 0)),
        out_shape=jax.ShapeDtypeStruct((N, H), jnp.float32),
    )


def _make_step_kernel(t):
    def body(x_ref, deg_ref, part_ref, hwsp_ref, we, be, wih, bih, whh, bhh,
             wg, bg, out_ref):
        dinv = _dinv_from_deg(deg_ref[...])
        p = part_ref[...]
        hprev = dinv * (p[0] + p[1] + hwsp_ref[...]) + bg[...]
        out_ref[...] = _gru_hws(x_ref[0], hprev, dinv, we[...], be[...],
                                wih[...], bih[...], whh[...], bhh[...], wg[...])

    in_specs = [
        pl.BlockSpec((1, BN, D_IN), lambda i, _t=t: (_t, i, 0)),
        pl.BlockSpec((NC, BN, H), lambda i: (0, i, 0)),
        pl.BlockSpec((NC, BN, H), lambda i: (0, i, 0)),
        pl.BlockSpec((BN, H), lambda i: (i, 0)),
    ] + _W_SPECS
    return pl.pallas_call(
        body,
        grid=(N // BN,),
        in_specs=in_specs,
        out_specs=pl.BlockSpec((BN, H), lambda i: (i, 0)),
        out_shape=jax.ShapeDtypeStruct((N, H), jnp.float32),
    )


def _make_final_kernel():
    def body(deg_ref, part_ref, hwsp_ref, bg, wfc, bfc, out_ref):
        dinv = _dinv_from_deg(deg_ref[...])
        p = part_ref[...]
        h = dinv * (p[0] + p[1] + hwsp_ref[...]) + bg[...]
        dn = (((1,), (1,)), ((), ()))
        out_ref[...] = lax.dot_general(
            h, wfc[...], dn, preferred_element_type=jnp.float32) + bfc[...]

    in_specs = [
        pl.BlockSpec((NC, BN, H), lambda i: (0, i, 0)),
        pl.BlockSpec((NC, BN, H), lambda i: (0, i, 0)),
        pl.BlockSpec((BN, H), lambda i: (i, 0)),
        pl.BlockSpec((1, H), lambda i: (0, 0)),
        pl.BlockSpec((D_OUT, H), lambda i: (0, 0)),
        pl.BlockSpec((1, D_OUT), lambda i: (0, 0)),
    ]
    return pl.pallas_call(
        body,
        grid=(N // BN,),
        in_specs=in_specs,
        out_specs=pl.BlockSpec((BN, D_OUT), lambda i: (i, 0)),
        out_shape=jax.ShapeDtypeStruct((N, D_OUT), jnp.float32),
    )


def kernel(x, edge_index, W_emb, b_emb, W_ih, W_hh, b_ih, b_hh, W_gcn, b_gcn,
           W_fc, b_fc):
    row3 = edge_index[0].reshape(NW, CPW, K)
    col3 = edge_index[1].reshape(NW, CPW, K)
    xT = jnp.transpose(x, (1, 0, 2))
    zeros_nh = jnp.zeros((NP, H), jnp.float32)
    ones_nh = jnp.ones((N, H), jnp.float32)
    be = b_emb.reshape(1, H)
    bih = b_ih.reshape(1, 3 * H)
    bhh = b_hh.reshape(1, 3 * H)
    bg = b_gcn.reshape(1, H)
    bfc = b_fc.reshape(1, D_OUT)

    scat = _make_scat_kernel()
    deg = scat(ones_nh, row3, col3, zeros_nh)

    hws = _make_step0_kernel()(
        xT, deg, W_emb, be, W_ih, bih, W_hh, bhh, W_gcn, bg)
    for t in range(T):
        part = scat(hws, row3, col3, zeros_nh)
        if t < T - 1:
            hws = _make_step_kernel(t + 1)(
                xT, deg, part, hws, W_emb, be, W_ih, bih, W_hh, bhh, W_gcn, bg)
    return _make_final_kernel()(deg, part, hws, bg, W_fc, bfc)
